# baseline (speedup 1.0000x reference)
import functools

import jax
import jax.numpy as jnp
from jax import lax
from jax.experimental import pallas as pl
from jax.experimental.pallas import tpu as pltpu
from jax.experimental.pallas import tpu_sc as plsc

H = 128
N = 327680
B = 16384

_NC, _NS = 2, 16
_CH = 128

_BLK = 2048


def _mid_body(t1r_ref, emb_ref, nc_ref, w2t_ref, b2_ref, q_ref, qb_ref, c_ref):
    emb = emb_ref[...]
    t2 = jnp.dot(
        emb.astype(jnp.bfloat16),
        w2t_ref[...].astype(jnp.bfloat16),
        preferred_element_type=jnp.float32,
    )
    pre = t1r_ref[...] + t2 + b2_ref[...]
    sig = jax.nn.sigmoid(pre)
    alpha = jnp.sum(sig * q_ref[...], axis=1, keepdims=True) + qb_ref[...]
    c_ref[...] = (alpha * nc_ref[...]) * emb


def _mid(t1_rep, emb, nc_col, W2t, b2_row, q_row, qb):
    grid = (N // _BLK,)
    return pl.pallas_call(
        _mid_body,
        grid=grid,
        in_specs=[
            pl.BlockSpec((_BLK, H), lambda i: (i, 0)),
            pl.BlockSpec((_BLK, H), lambda i: (i, 0)),
            pl.BlockSpec((_BLK, 1), lambda i: (i, 0)),
            pl.BlockSpec((H, H), lambda i: (0, 0)),
            pl.BlockSpec((1, H), lambda i: (0, 0)),
            pl.BlockSpec((1, H), lambda i: (0, 0)),
            pl.BlockSpec((1, 1), lambda i: (0, 0)),
        ],
        out_specs=pl.BlockSpec((_BLK, H), lambda i: (i, 0)),
        out_shape=jax.ShapeDtypeStruct((N, H), jnp.float32),
    )(t1_rep, emb, nc_col, W2t, b2_row, q_row, qb)


_SC_MESH = plsc.VectorSubcoreMesh(core_axis_name="c", subcore_axis_name="s")


_NW = _NC * _NS
_SEG_W = B // (2 * _NW)
_NB = 5


def _make_segsum(h):
    @functools.partial(
        pl.kernel,
        out_type=jax.ShapeDtypeStruct((B // 2, H), jnp.float32),
        mesh=_SC_MESH,
        scratch_types=[
            pltpu.VMEM((_NB, _CH, H), jnp.float32),
            pltpu.VMEM((_NB, _CH), jnp.int32),
            pltpu.VMEM((16,), jnp.int32),
            pltpu.VMEM((16,), jnp.int32),
            pltpu.VMEM_SHARED((_NS * _SEG_W, H), jnp.float32),
            pltpu.SemaphoreType.DMA,
            pltpu.SemaphoreType.DMA,
        ],
    )
    def _segsum_kernel(c_hbm, b_hbm, cntl_hbm, zero_hbm, sg_hbm, rows_v, bidx_v, lo_v, hi_v, acc_sh, rsem, ssem):
        c = lax.axis_index("c")
        s = lax.axis_index("s")
        w = s * _NC + c
        segstart = h * (B // 2) + w * _SEG_W
        accbase = s * _SEG_W
        for j in range(_SEG_W // _CH):
            pltpu.async_copy(zero_hbm, acc_sh.at[pl.ds(accbase + j * _CH, _CH)], ssem)
        pltpu.async_copy(cntl_hbm.at[pl.ds(segstart, 16)], lo_v, rsem)
        pltpu.async_copy(cntl_hbm.at[pl.ds(segstart + _SEG_W, 16)], hi_v, rsem)
        pltpu.make_async_copy(cntl_hbm.at[pl.ds(segstart, 16)], lo_v, rsem).wait()
        pltpu.make_async_copy(cntl_hbm.at[pl.ds(segstart, 16)], hi_v, rsem).wait()
        lo = lo_v[pl.ds(0, 16)][0]
        hi = hi_v[pl.ds(0, 16)][0]
        for j in range(_SEG_W // _CH):
            pltpu.make_async_copy(zero_hbm, acc_sh.at[pl.ds(accbase + j * _CH, _CH)], ssem).wait()

        lo_al = (lo // _CH) * _CH
        nch = (hi - lo_al + _CH - 1) // _CH
        nchg = (nch + _NB - 1) // _NB
        iot = lax.broadcasted_iota(jnp.int32, (16,), 0)

        def _scat(b):
            return pltpu.make_async_copy(
                rows_v.at[b],
                acc_sh.at[plsc.Indices(bidx_v.at[b], ignored_value=-1)],
                ssem,
            )

        @pl.loop(0, nchg)
        def _grp(j):
            @pl.when(j > 0)
            def _():
                for b in range(_NB):
                    _scat(b).wait()

            for b in range(_NB):
                base = lo_al + (j * _NB + b) * _CH

                @pl.when(base < hi)
                def _():
                    pltpu.async_copy(c_hbm.at[pl.ds(base, _CH)], rows_v.at[b], rsem)
                    pltpu.async_copy(b_hbm.at[pl.ds(base, _CH)], bidx_v.at[b], rsem)
            for b in range(_NB):
                base = lo_al + (j * _NB + b) * _CH

                @pl.when(base < hi)
                def _():
                    pltpu.make_async_copy(c_hbm.at[pl.ds(base, _CH)], rows_v.at[b], rsem).wait()
                    pltpu.make_async_copy(b_hbm.at[pl.ds(base, _CH)], bidx_v.at[b], rsem).wait()
            for b in range(_NB):
                base = lo_al + (j * _NB + b) * _CH
                for g in range(_CH // 16):
                    v = bidx_v[b, pl.ds(g * 16, 16)]
                    rowid = base + g * 16 + iot
                    valid = (rowid >= lo) & (rowid < hi)
                    bidx_v[b, pl.ds(g * 16, 16)] = jnp.where(
                        valid, v - segstart + accbase, -1
                    )
                pltpu.async_copy(
                    rows_v.at[b],
                    acc_sh.at[plsc.Indices(bidx_v.at[b], ignored_value=-1)],
                    ssem,
                    add=True,
                )

        @pl.when(nchg > 0)
        def _():
            for b in range(_NB):
                _scat(b).wait()

        pltpu.async_copy(
            acc_sh.at[pl.ds(accbase, _SEG_W)],
            sg_hbm.at[pl.ds(w * _SEG_W, _SEG_W)],
            rsem,
        ).wait()

    return _segsum_kernel


_segsum_lo = _make_segsum(0)
_segsum_hi = _make_segsum(1)


@functools.partial(
    pl.kernel,
    out_type=jax.ShapeDtypeStruct((_NC, B), jnp.int32),
    mesh=_SC_MESH,
    scratch_types=[
        pltpu.VMEM((8, _CH), jnp.int32),
        pltpu.VMEM((8, _CH), jnp.int32),
        pltpu.VMEM((B // _NS,), jnp.int32),
        pltpu.VMEM_SHARED((B,), jnp.int32),
        pltpu.SemaphoreType.DMA,
    ],
)
def _endscan_kernel(b_hbm, ev_hbm, part_hbm, bidx_v, bval_v, z_v, sh, sem):
    c = lax.axis_index("c")
    s = lax.axis_index("s")
    wid = s * _NC + c
    colbase = s * (B // _NS)

    @pl.loop(0, B // _NS // 16)
    def _zero(i):
        z_v[pl.ds(i * 16, 16)] = jnp.zeros((16,), jnp.int32)

    pltpu.async_copy(z_v, sh.at[pl.ds(colbase, B // _NS)], sem).wait()
    plsc.subcore_barrier()

    nw = N // (_NC * _NS)
    base0 = wid * nw

    @pl.loop(0, nw // (8 * _CH))
    def _chunk(kk):
        base = base0 + kk * 8 * _CH
        for i in range(8):
            pltpu.async_copy(b_hbm.at[pl.ds(base + i * _CH, _CH)], bidx_v.at[i], sem)
            pltpu.async_copy(ev_hbm.at[pl.ds(base + i * _CH, _CH)], bval_v.at[i], sem)
        for i in range(8):
            pltpu.make_async_copy(b_hbm.at[pl.ds(base, _CH)], bidx_v.at[i], sem).wait()
            pltpu.make_async_copy(ev_hbm.at[pl.ds(base, _CH)], bval_v.at[i], sem).wait()
        for i in range(8):
            for g in range(_CH // 16):
                e = bval_v[i, pl.ds(g * 16, 16)]
                bv = bidx_v[i, pl.ds(g * 16, 16)]
                bidx_v[i, pl.ds(g * 16, 16)] = jnp.where(e > 0, bv, -1)
            pltpu.async_copy(
                bval_v.at[i], sh.at[plsc.Indices(bidx_v.at[i], ignored_value=-1)], sem
            )
        for i in range(8):
            pltpu.make_async_copy(
                bval_v.at[i], sh.at[plsc.Indices(bidx_v.at[i], ignored_value=-1)], sem
            ).wait()

    plsc.subcore_barrier()
    pltpu.async_copy(
        sh.at[pl.ds(colbase, B // _NS)],
        part_hbm.at[c].at[pl.ds(colbase, B // _NS)],
        sem,
    ).wait()


def _make_gather(n_rows, n_src):
    nw = n_rows // (_NC * _NS)
    nch = nw // _CH
    step = min(5, nch)
    while nch % step:
        step -= 1

    @functools.partial(
        pl.kernel,
        out_type=jax.ShapeDtypeStruct((n_rows, H), jnp.float32),
        mesh=_SC_MESH,
        scratch_types=[
            pltpu.VMEM((step, _CH, H), jnp.float32),
            pltpu.VMEM((nw,), jnp.int32),
            pltpu.SemaphoreType.DMA,
            pltpu.SemaphoreType.DMA,
        ],
    )
    def _gather_kernel(src_hbm, idx_hbm, out_hbm, rows_v, idx_v, gsem, wsem):
        c = lax.axis_index("c")
        s = lax.axis_index("s")
        wid = s * _NC + c
        base0 = wid * nw
        pltpu.async_copy(idx_hbm.at[pl.ds(base0, nw)], idx_v, gsem).wait()

        @pl.loop(0, nch, step=step)
        def _chunk(k0):
            @pl.when(k0 > 0)
            def _():
                for b in range(step):
                    pltpu.make_async_copy(
                        rows_v.at[b], out_hbm.at[pl.ds(base0, _CH)], wsem
                    ).wait()

            gds = []
            for b in range(step):
                gds.append(pltpu.async_copy(
                    src_hbm.at[plsc.Indices(idx_v.at[pl.ds((k0 + b) * _CH, _CH)])],
                    rows_v.at[b],
                    gsem,
                ))
            for gd in gds:
                gd.wait()
            for b in range(step):
                pltpu.async_copy(
                    rows_v.at[b],
                    out_hbm.at[pl.ds(base0 + (k0 + b) * _CH, _CH)],
                    wsem,
                )

        for b in range(step):
            pltpu.make_async_copy(
                rows_v.at[b], out_hbm.at[pl.ds(base0, _CH)], wsem
            ).wait()

    return _gather_kernel


_gather_n = _make_gather(N, B)
_gather_b = _make_gather(B, N)


def _cummax_body(p_ref, o_ref):
    x = p_ref[0] + p_ref[1]
    li = lax.broadcasted_iota(jnp.int32, (128, 128), 1)
    ri = lax.broadcasted_iota(jnp.int32, (128, 128), 0)
    for k in (1, 2, 4, 8, 16, 32, 64):
        x = jnp.maximum(x, jnp.where(li >= k, pltpu.roll(x, k, 1), 0))
    col = jnp.max(x, axis=1, keepdims=True)
    y = jnp.where(ri >= 1, pltpu.roll(jnp.broadcast_to(col, (128, 128)), 1, 0), 0)
    for k in (1, 2, 4, 8, 16, 32, 64):
        y = jnp.maximum(y, jnp.where(ri >= k, pltpu.roll(y, k, 0), 0))
    o_ref[...] = jnp.maximum(x, y)


def _cummax(part3d):
    return pl.pallas_call(
        _cummax_body,
        out_shape=jax.ShapeDtypeStruct((128, 128), jnp.int32),
    )(part3d)


_SB = 2048


def _t1_body(v_ref, w_ref, b_ref, o_ref):
    o_ref[...] = (
        jnp.dot(
            v_ref[...].astype(jnp.bfloat16),
            w_ref[...].astype(jnp.bfloat16),
            preferred_element_type=jnp.float32,
        )
        + b_ref[...]
    )


def _t1_mm(v_n, W1t, b1row):
    return pl.pallas_call(
        _t1_body,
        grid=(B // _SB,),
        in_specs=[
            pl.BlockSpec((_SB, H), lambda i: (i, 0)),
            pl.BlockSpec((H, H), lambda i: (0, 0)),
            pl.BlockSpec((1, H), lambda i: (0, 0)),
        ],
        out_specs=pl.BlockSpec((_SB, H), lambda i: (i, 0)),
        out_shape=jax.ShapeDtypeStruct((B, H), jnp.float32),
    )(v_n, W1t, b1row)


def _sh_body(v_ref, g_ref, wa_ref, wb_ref, b_ref, o_ref):
    o_ref[...] = (
        jnp.dot(
            v_ref[...].astype(jnp.bfloat16),
            wa_ref[...].astype(jnp.bfloat16),
            preferred_element_type=jnp.float32,
        )
        + jnp.dot(
            g_ref[...].astype(jnp.bfloat16),
            wb_ref[...].astype(jnp.bfloat16),
            preferred_element_type=jnp.float32,
        )
        + b_ref[...]
    )


def _sh_mm(v_n, s_g, W3at, W3bt, b3row):
    return pl.pallas_call(
        _sh_body,
        grid=(B // _SB,),
        in_specs=[
            pl.BlockSpec((_SB, H), lambda i: (i, 0)),
            pl.BlockSpec((_SB, H), lambda i: (i, 0)),
            pl.BlockSpec((H, H), lambda i: (0, 0)),
            pl.BlockSpec((H, H), lambda i: (0, 0)),
            pl.BlockSpec((1, H), lambda i: (0, 0)),
        ],
        out_specs=pl.BlockSpec((_SB, H), lambda i: (i, 0)),
        out_shape=jax.ShapeDtypeStruct((B, H), jnp.float32),
    )(v_n, s_g, W3at, W3bt, b3row)


def kernel(node_embedding, batch, num_count, W1_w, W1_b, W2_w, W2_b, q_w, q_b, W3_w, W3_b):
    batch32 = batch.astype(jnp.int32)
    nxt = jnp.concatenate([batch32[1:], jnp.full((1,), -1, jnp.int32)])
    endv = jnp.where(batch32 != nxt, jnp.arange(N, dtype=jnp.int32) + 1, 0)
    part = _endscan_kernel(batch32, endv)
    cnt = _cummax(part.reshape(_NC, 128, 128)).reshape(B)
    ends = jnp.clip(cnt - 1, 0, N - 1)
    v_n = _gather_b(node_embedding, ends)
    t1 = _t1_mm(v_n, W1_w.T, W1_b.reshape(1, H))
    t1_rep = _gather_n(t1, batch32)
    c = _mid(
        t1_rep,
        node_embedding,
        num_count.reshape(N, 1),
        W2_w.T,
        W2_b.reshape(1, H),
        q_w.reshape(1, H),
        q_b.reshape(1, 1),
    )
    zz = jnp.zeros((_CH, H), jnp.float32)
    cntl = jnp.concatenate(
        [jnp.zeros((1,), jnp.int32), cnt, jnp.full((15,), N, jnp.int32)]
    )
    s_g = jnp.concatenate(
        [_segsum_lo(c, batch32, cntl, zz), _segsum_hi(c, batch32, cntl, zz)]
    )
    s_h = _sh_mm(v_n, s_g, W3_w[:, :H].T, W3_w[:, H:].T, W3_b.reshape(1, H))
    return s_h


# speedup vs baseline: 1.0707x; 1.0707x over previous
import functools

import jax
import jax.numpy as jnp
from jax import lax
from jax.experimental import pallas as pl
from jax.experimental.pallas import tpu as pltpu
from jax.experimental.pallas import tpu_sc as plsc

H = 128
N = 327680
B = 16384

_NC, _NS = 2, 16
_CH = 128

_BLK = 4096


def _mid_body(t1r_ref, emb_ref, nc_ref, w2t_ref, b2_ref, q_ref, qb_ref, c_ref):
    emb = emb_ref[...]
    t2 = jnp.dot(
        emb.astype(jnp.bfloat16),
        w2t_ref[...].astype(jnp.bfloat16),
        preferred_element_type=jnp.float32,
    )
    pre = t1r_ref[...] + t2 + b2_ref[...]
    sig = jax.nn.sigmoid(pre)
    alpha = jnp.sum(sig * q_ref[...], axis=1, keepdims=True) + qb_ref[...]
    c_ref[...] = (alpha * nc_ref[...]) * emb


def _mid(t1_rep, emb, nc_col, W2t, b2_row, q_row, qb):
    grid = (N // _BLK,)
    return pl.pallas_call(
        _mid_body,
        grid=grid,
        in_specs=[
            pl.BlockSpec((_BLK, H), lambda i: (i, 0)),
            pl.BlockSpec((_BLK, H), lambda i: (i, 0)),
            pl.BlockSpec((_BLK, 1), lambda i: (i, 0)),
            pl.BlockSpec((H, H), lambda i: (0, 0)),
            pl.BlockSpec((1, H), lambda i: (0, 0)),
            pl.BlockSpec((1, H), lambda i: (0, 0)),
            pl.BlockSpec((1, 1), lambda i: (0, 0)),
        ],
        out_specs=pl.BlockSpec((_BLK, H), lambda i: (i, 0)),
        out_shape=jax.ShapeDtypeStruct((N, H), jnp.float32),
    )(t1_rep, emb, nc_col, W2t, b2_row, q_row, qb)


_SC_MESH = plsc.VectorSubcoreMesh(core_axis_name="c", subcore_axis_name="s")


_NW = _NC * _NS
_SEG_W = B // (2 * _NW)
_NB = 5


def _make_segsum(h):
    @functools.partial(
        pl.kernel,
        out_type=jax.ShapeDtypeStruct((B // 2, H), jnp.float32),
        mesh=_SC_MESH,
        scratch_types=[
            pltpu.VMEM((_NB, _CH, H), jnp.float32),
            pltpu.VMEM((_NB, _CH), jnp.int32),
            pltpu.VMEM((16,), jnp.int32),
            pltpu.VMEM((16,), jnp.int32),
            pltpu.VMEM_SHARED((_NS * _SEG_W, H), jnp.float32),
            pltpu.SemaphoreType.DMA,
            pltpu.SemaphoreType.DMA,
        ],
    )
    def _segsum_kernel(c_hbm, b_hbm, cntl_hbm, zero_hbm, sg_hbm, rows_v, bidx_v, lo_v, hi_v, acc_sh, rsem, ssem):
        c = lax.axis_index("c")
        s = lax.axis_index("s")
        w = s * _NC + c
        segstart = h * (B // 2) + w * _SEG_W
        accbase = s * _SEG_W
        for j in range(_SEG_W // _CH):
            pltpu.async_copy(zero_hbm, acc_sh.at[pl.ds(accbase + j * _CH, _CH)], ssem)
        pltpu.async_copy(cntl_hbm.at[pl.ds(segstart, 16)], lo_v, rsem)
        pltpu.async_copy(cntl_hbm.at[pl.ds(segstart + _SEG_W, 16)], hi_v, rsem)
        pltpu.make_async_copy(cntl_hbm.at[pl.ds(segstart, 16)], lo_v, rsem).wait()
        pltpu.make_async_copy(cntl_hbm.at[pl.ds(segstart, 16)], hi_v, rsem).wait()
        lo = lo_v[pl.ds(0, 16)][0]
        hi = hi_v[pl.ds(0, 16)][0]
        for j in range(_SEG_W // _CH):
            pltpu.make_async_copy(zero_hbm, acc_sh.at[pl.ds(accbase + j * _CH, _CH)], ssem).wait()

        lo_al = (lo // _CH) * _CH
        nch = (hi - lo_al + _CH - 1) // _CH
        nchg = (nch + _NB - 1) // _NB
        iot = lax.broadcasted_iota(jnp.int32, (16,), 0)

        def _scat(b):
            return pltpu.make_async_copy(
                rows_v.at[b],
                acc_sh.at[plsc.Indices(bidx_v.at[b], ignored_value=-1)],
                ssem,
            )

        @pl.loop(0, nchg)
        def _grp(j):
            @pl.when(j > 0)
            def _():
                for b in range(_NB):
                    _scat(b).wait()

            for b in range(_NB):
                base = lo_al + (j * _NB + b) * _CH

                @pl.when(base < hi)
                def _():
                    pltpu.async_copy(c_hbm.at[pl.ds(base, _CH)], rows_v.at[b], rsem)
                    pltpu.async_copy(b_hbm.at[pl.ds(base, _CH)], bidx_v.at[b], rsem)
            for b in range(_NB):
                base = lo_al + (j * _NB + b) * _CH

                @pl.when(base < hi)
                def _():
                    pltpu.make_async_copy(c_hbm.at[pl.ds(base, _CH)], rows_v.at[b], rsem).wait()
                    pltpu.make_async_copy(b_hbm.at[pl.ds(base, _CH)], bidx_v.at[b], rsem).wait()
            for b in range(_NB):
                base = lo_al + (j * _NB + b) * _CH
                for g in range(_CH // 16):
                    v = bidx_v[b, pl.ds(g * 16, 16)]
                    rowid = base + g * 16 + iot
                    valid = (rowid >= lo) & (rowid < hi)
                    bidx_v[b, pl.ds(g * 16, 16)] = jnp.where(
                        valid, v - segstart + accbase, -1
                    )
                pltpu.async_copy(
                    rows_v.at[b],
                    acc_sh.at[plsc.Indices(bidx_v.at[b], ignored_value=-1)],
                    ssem,
                    add=True,
                )

        @pl.when(nchg > 0)
        def _():
            for b in range(_NB):
                _scat(b).wait()

        pltpu.async_copy(
            acc_sh.at[pl.ds(accbase, _SEG_W)],
            sg_hbm.at[pl.ds(w * _SEG_W, _SEG_W)],
            rsem,
        ).wait()

    return _segsum_kernel


_segsum_lo = _make_segsum(0)
_segsum_hi = _make_segsum(1)


@functools.partial(
    pl.kernel,
    out_type=jax.ShapeDtypeStruct((_NC, B), jnp.int32),
    mesh=_SC_MESH,
    scratch_types=[
        pltpu.VMEM((8, _CH), jnp.int32),
        pltpu.VMEM((8, _CH), jnp.int32),
        pltpu.VMEM((B // _NS,), jnp.int32),
        pltpu.VMEM_SHARED((B,), jnp.int32),
        pltpu.SemaphoreType.DMA,
    ],
)
def _endscan_kernel(b_hbm, ev_hbm, part_hbm, bidx_v, bval_v, z_v, sh, sem):
    c = lax.axis_index("c")
    s = lax.axis_index("s")
    wid = s * _NC + c
    colbase = s * (B // _NS)

    @pl.loop(0, B // _NS // 16)
    def _zero(i):
        z_v[pl.ds(i * 16, 16)] = jnp.zeros((16,), jnp.int32)

    pltpu.async_copy(z_v, sh.at[pl.ds(colbase, B // _NS)], sem).wait()
    plsc.subcore_barrier()

    nw = N // (_NC * _NS)
    base0 = wid * nw

    @pl.loop(0, nw // (8 * _CH))
    def _chunk(kk):
        base = base0 + kk * 8 * _CH
        for i in range(8):
            pltpu.async_copy(b_hbm.at[pl.ds(base + i * _CH, _CH)], bidx_v.at[i], sem)
            pltpu.async_copy(ev_hbm.at[pl.ds(base + i * _CH, _CH)], bval_v.at[i], sem)
        for i in range(8):
            pltpu.make_async_copy(b_hbm.at[pl.ds(base, _CH)], bidx_v.at[i], sem).wait()
            pltpu.make_async_copy(ev_hbm.at[pl.ds(base, _CH)], bval_v.at[i], sem).wait()
        for i in range(8):
            for g in range(_CH // 16):
                e = bval_v[i, pl.ds(g * 16, 16)]
                bv = bidx_v[i, pl.ds(g * 16, 16)]
                bidx_v[i, pl.ds(g * 16, 16)] = jnp.where(e > 0, bv, -1)
            pltpu.async_copy(
                bval_v.at[i], sh.at[plsc.Indices(bidx_v.at[i], ignored_value=-1)], sem
            )
        for i in range(8):
            pltpu.make_async_copy(
                bval_v.at[i], sh.at[plsc.Indices(bidx_v.at[i], ignored_value=-1)], sem
            ).wait()

    plsc.subcore_barrier()
    pltpu.async_copy(
        sh.at[pl.ds(colbase, B // _NS)],
        part_hbm.at[c].at[pl.ds(colbase, B // _NS)],
        sem,
    ).wait()


def _make_gather(n_rows, n_src):
    nw = n_rows // (_NC * _NS)
    nch = nw // _CH
    step = min(5, nch)
    while nch % step:
        step -= 1

    @functools.partial(
        pl.kernel,
        out_type=jax.ShapeDtypeStruct((n_rows, H), jnp.float32),
        mesh=_SC_MESH,
        scratch_types=[
            pltpu.VMEM((step, _CH, H), jnp.float32),
            pltpu.VMEM((nw,), jnp.int32),
            pltpu.SemaphoreType.DMA,
            pltpu.SemaphoreType.DMA,
        ],
    )
    def _gather_kernel(src_hbm, idx_hbm, out_hbm, rows_v, idx_v, gsem, wsem):
        c = lax.axis_index("c")
        s = lax.axis_index("s")
        wid = s * _NC + c
        base0 = wid * nw
        pltpu.async_copy(idx_hbm.at[pl.ds(base0, nw)], idx_v, gsem).wait()

        @pl.loop(0, nch, step=step)
        def _chunk(k0):
            @pl.when(k0 > 0)
            def _():
                for b in range(step):
                    pltpu.make_async_copy(
                        rows_v.at[b], out_hbm.at[pl.ds(base0, _CH)], wsem
                    ).wait()

            gds = []
            for b in range(step):
                gds.append(pltpu.async_copy(
                    src_hbm.at[plsc.Indices(idx_v.at[pl.ds((k0 + b) * _CH, _CH)])],
                    rows_v.at[b],
                    gsem,
                ))
            for gd in gds:
                gd.wait()
            for b in range(step):
                pltpu.async_copy(
                    rows_v.at[b],
                    out_hbm.at[pl.ds(base0 + (k0 + b) * _CH, _CH)],
                    wsem,
                )

        for b in range(step):
            pltpu.make_async_copy(
                rows_v.at[b], out_hbm.at[pl.ds(base0, _CH)], wsem
            ).wait()

    return _gather_kernel


_gather_n = _make_gather(N, B)
_gather_b = _make_gather(B, N)


def _cummax_body(p_ref, o_ref):
    x = p_ref[0] + p_ref[1]
    li = lax.broadcasted_iota(jnp.int32, (128, 128), 1)
    ri = lax.broadcasted_iota(jnp.int32, (128, 128), 0)
    for k in (1, 2, 4, 8, 16, 32, 64):
        x = jnp.maximum(x, jnp.where(li >= k, pltpu.roll(x, k, 1), 0))
    col = jnp.max(x, axis=1, keepdims=True)
    y = jnp.where(ri >= 1, pltpu.roll(jnp.broadcast_to(col, (128, 128)), 1, 0), 0)
    for k in (1, 2, 4, 8, 16, 32, 64):
        y = jnp.maximum(y, jnp.where(ri >= k, pltpu.roll(y, k, 0), 0))
    o_ref[...] = jnp.maximum(x, y)


def _cummax(part3d):
    return pl.pallas_call(
        _cummax_body,
        out_shape=jax.ShapeDtypeStruct((128, 128), jnp.int32),
    )(part3d)


_SB = 2048


def _t1_body(v_ref, w_ref, b_ref, o_ref):
    o_ref[...] = (
        jnp.dot(
            v_ref[...].astype(jnp.bfloat16),
            w_ref[...].astype(jnp.bfloat16),
            preferred_element_type=jnp.float32,
        )
        + b_ref[...]
    )


def _t1_mm(v_n, W1t, b1row):
    return pl.pallas_call(
        _t1_body,
        grid=(B // _SB,),
        in_specs=[
            pl.BlockSpec((_SB, H), lambda i: (i, 0)),
            pl.BlockSpec((H, H), lambda i: (0, 0)),
            pl.BlockSpec((1, H), lambda i: (0, 0)),
        ],
        out_specs=pl.BlockSpec((_SB, H), lambda i: (i, 0)),
        out_shape=jax.ShapeDtypeStruct((B, H), jnp.float32),
    )(v_n, W1t, b1row)


def _sh_body(v_ref, g_ref, wa_ref, wb_ref, b_ref, o_ref):
    o_ref[...] = (
        jnp.dot(
            v_ref[...].astype(jnp.bfloat16),
            wa_ref[...].astype(jnp.bfloat16),
            preferred_element_type=jnp.float32,
        )
        + jnp.dot(
            g_ref[...].astype(jnp.bfloat16),
            wb_ref[...].astype(jnp.bfloat16),
            preferred_element_type=jnp.float32,
        )
        + b_ref[...]
    )


def _sh_mm(v_n, s_g, W3at, W3bt, b3row):
    return pl.pallas_call(
        _sh_body,
        grid=(B // _SB,),
        in_specs=[
            pl.BlockSpec((_SB, H), lambda i: (i, 0)),
            pl.BlockSpec((_SB, H), lambda i: (i, 0)),
            pl.BlockSpec((H, H), lambda i: (0, 0)),
            pl.BlockSpec((H, H), lambda i: (0, 0)),
            pl.BlockSpec((1, H), lambda i: (0, 0)),
        ],
        out_specs=pl.BlockSpec((_SB, H), lambda i: (i, 0)),
        out_shape=jax.ShapeDtypeStruct((B, H), jnp.float32),
    )(v_n, s_g, W3at, W3bt, b3row)


def kernel(node_embedding, batch, num_count, W1_w, W1_b, W2_w, W2_b, q_w, q_b, W3_w, W3_b):
    batch32 = batch.astype(jnp.int32)
    nxt = jnp.concatenate([batch32[1:], jnp.full((1,), -1, jnp.int32)])
    endv = jnp.where(batch32 != nxt, jnp.arange(N, dtype=jnp.int32) + 1, 0)
    part = _endscan_kernel(batch32, endv)
    cnt = _cummax(part.reshape(_NC, 128, 128)).reshape(B)
    ends = jnp.clip(cnt - 1, 0, N - 1)
    v_n = _gather_b(node_embedding, ends)
    t1 = _t1_mm(v_n, W1_w.T, W1_b.reshape(1, H))
    t1_rep = _gather_n(t1, batch32)
    c = _mid(
        t1_rep,
        node_embedding,
        num_count.reshape(N, 1),
        W2_w.T,
        W2_b.reshape(1, H),
        q_w.reshape(1, H),
        q_b.reshape(1, 1),
    )
    zz = jnp.zeros((_CH, H), jnp.float32)
    cntl = jnp.concatenate(
        [jnp.zeros((1,), jnp.int32), cnt, jnp.full((15,), N, jnp.int32)]
    )
    s_g = jnp.concatenate(
        [_segsum_lo(c, batch32, cntl, zz), _segsum_hi(c, batch32, cntl, zz)]
    )
    s_h = _sh_mm(v_n, s_g, W3_w[:, :H].T, W3_w[:, H:].T, W3_b.reshape(1, H))
    return s_h


# speedup vs baseline: 1.0916x; 1.0195x over previous
import functools

import jax
import jax.numpy as jnp
from jax import lax
from jax.experimental import pallas as pl
from jax.experimental.pallas import tpu as pltpu
from jax.experimental.pallas import tpu_sc as plsc

H = 128
N = 327680
B = 16384

_NC, _NS = 2, 16
_CH = 128

_BLK = 8192


def _mid_body(t1r_ref, emb_ref, nc_ref, w2t_ref, b2_ref, q_ref, qb_ref, c_ref):
    emb = emb_ref[...]
    t2 = jnp.dot(
        emb.astype(jnp.bfloat16),
        w2t_ref[...].astype(jnp.bfloat16),
        preferred_element_type=jnp.float32,
    )
    pre = t1r_ref[...] + t2 + b2_ref[...]
    sig = jax.nn.sigmoid(pre)
    alpha = jnp.sum(sig * q_ref[...], axis=1, keepdims=True) + qb_ref[...]
    c_ref[...] = (alpha * nc_ref[...]) * emb


def _mid(t1_rep, emb, nc_col, W2t, b2_row, q_row, qb):
    grid = (N // _BLK,)
    return pl.pallas_call(
        _mid_body,
        grid=grid,
        in_specs=[
            pl.BlockSpec((_BLK, H), lambda i: (i, 0)),
            pl.BlockSpec((_BLK, H), lambda i: (i, 0)),
            pl.BlockSpec((_BLK, 1), lambda i: (i, 0)),
            pl.BlockSpec((H, H), lambda i: (0, 0)),
            pl.BlockSpec((1, H), lambda i: (0, 0)),
            pl.BlockSpec((1, H), lambda i: (0, 0)),
            pl.BlockSpec((1, 1), lambda i: (0, 0)),
        ],
        out_specs=pl.BlockSpec((_BLK, H), lambda i: (i, 0)),
        out_shape=jax.ShapeDtypeStruct((N, H), jnp.float32),
    )(t1_rep, emb, nc_col, W2t, b2_row, q_row, qb)


_SC_MESH = plsc.VectorSubcoreMesh(core_axis_name="c", subcore_axis_name="s")


_NW = _NC * _NS
_SEG_W = B // (2 * _NW)
_NB = 5


def _make_segsum(h):
    @functools.partial(
        pl.kernel,
        out_type=jax.ShapeDtypeStruct((B // 2, H), jnp.float32),
        mesh=_SC_MESH,
        scratch_types=[
            pltpu.VMEM((_NB, _CH, H), jnp.float32),
            pltpu.VMEM((_NB, _CH), jnp.int32),
            pltpu.VMEM((16,), jnp.int32),
            pltpu.VMEM((16,), jnp.int32),
            pltpu.VMEM_SHARED((_NS * _SEG_W, H), jnp.float32),
            pltpu.SemaphoreType.DMA,
            pltpu.SemaphoreType.DMA,
        ],
    )
    def _segsum_kernel(c_hbm, b_hbm, cntl_hbm, zero_hbm, sg_hbm, rows_v, bidx_v, lo_v, hi_v, acc_sh, rsem, ssem):
        c = lax.axis_index("c")
        s = lax.axis_index("s")
        w = s * _NC + c
        segstart = h * (B // 2) + w * _SEG_W
        accbase = s * _SEG_W
        for j in range(_SEG_W // _CH):
            pltpu.async_copy(zero_hbm, acc_sh.at[pl.ds(accbase + j * _CH, _CH)], ssem)
        pltpu.async_copy(cntl_hbm.at[pl.ds(segstart, 16)], lo_v, rsem)
        pltpu.async_copy(cntl_hbm.at[pl.ds(segstart + _SEG_W, 16)], hi_v, rsem)
        pltpu.make_async_copy(cntl_hbm.at[pl.ds(segstart, 16)], lo_v, rsem).wait()
        pltpu.make_async_copy(cntl_hbm.at[pl.ds(segstart, 16)], hi_v, rsem).wait()
        lo = lo_v[pl.ds(0, 16)][0]
        hi = hi_v[pl.ds(0, 16)][0]
        for j in range(_SEG_W // _CH):
            pltpu.make_async_copy(zero_hbm, acc_sh.at[pl.ds(accbase + j * _CH, _CH)], ssem).wait()

        lo_al = (lo // _CH) * _CH
        nch = (hi - lo_al + _CH - 1) // _CH
        nchg = (nch + _NB - 1) // _NB
        iot = lax.broadcasted_iota(jnp.int32, (16,), 0)

        def _scat(b):
            return pltpu.make_async_copy(
                rows_v.at[b],
                acc_sh.at[plsc.Indices(bidx_v.at[b], ignored_value=-1)],
                ssem,
            )

        @pl.loop(0, nchg)
        def _grp(j):
            @pl.when(j > 0)
            def _():
                for b in range(_NB):
                    _scat(b).wait()

            for b in range(_NB):
                base = lo_al + (j * _NB + b) * _CH

                @pl.when(base < hi)
                def _():
                    pltpu.async_copy(c_hbm.at[pl.ds(base, _CH)], rows_v.at[b], rsem)
                    pltpu.async_copy(b_hbm.at[pl.ds(base, _CH)], bidx_v.at[b], rsem)
            for b in range(_NB):
                base = lo_al + (j * _NB + b) * _CH

                @pl.when(base < hi)
                def _():
                    pltpu.make_async_copy(c_hbm.at[pl.ds(base, _CH)], rows_v.at[b], rsem).wait()
                    pltpu.make_async_copy(b_hbm.at[pl.ds(base, _CH)], bidx_v.at[b], rsem).wait()
            for b in range(_NB):
                base = lo_al + (j * _NB + b) * _CH
                for g in range(_CH // 16):
                    v = bidx_v[b, pl.ds(g * 16, 16)]
                    rowid = base + g * 16 + iot
                    valid = (rowid >= lo) & (rowid < hi)
                    bidx_v[b, pl.ds(g * 16, 16)] = jnp.where(
                        valid, v - segstart + accbase, -1
                    )
                pltpu.async_copy(
                    rows_v.at[b],
                    acc_sh.at[plsc.Indices(bidx_v.at[b], ignored_value=-1)],
                    ssem,
                    add=True,
                )

        @pl.when(nchg > 0)
        def _():
            for b in range(_NB):
                _scat(b).wait()

        pltpu.async_copy(
            acc_sh.at[pl.ds(accbase, _SEG_W)],
            sg_hbm.at[pl.ds(w * _SEG_W, _SEG_W)],
            rsem,
        ).wait()

    return _segsum_kernel


_segsum_lo = _make_segsum(0)
_segsum_hi = _make_segsum(1)


@functools.partial(
    pl.kernel,
    out_type=jax.ShapeDtypeStruct((_NC, B), jnp.int32),
    mesh=_SC_MESH,
    scratch_types=[
        pltpu.VMEM((8, _CH), jnp.int32),
        pltpu.VMEM((8, _CH), jnp.int32),
        pltpu.VMEM((B // _NS,), jnp.int32),
        pltpu.VMEM_SHARED((B,), jnp.int32),
        pltpu.SemaphoreType.DMA,
    ],
)
def _endscan_kernel(b_hbm, ev_hbm, part_hbm, bidx_v, bval_v, z_v, sh, sem):
    c = lax.axis_index("c")
    s = lax.axis_index("s")
    wid = s * _NC + c
    colbase = s * (B // _NS)

    @pl.loop(0, B // _NS // 16)
    def _zero(i):
        z_v[pl.ds(i * 16, 16)] = jnp.zeros((16,), jnp.int32)

    pltpu.async_copy(z_v, sh.at[pl.ds(colbase, B // _NS)], sem).wait()
    plsc.subcore_barrier()

    nw = N // (_NC * _NS)
    base0 = wid * nw

    @pl.loop(0, nw // (8 * _CH))
    def _chunk(kk):
        base = base0 + kk * 8 * _CH
        for i in range(8):
            pltpu.async_copy(b_hbm.at[pl.ds(base + i * _CH, _CH)], bidx_v.at[i], sem)
            pltpu.async_copy(ev_hbm.at[pl.ds(base + i * _CH, _CH)], bval_v.at[i], sem)
        for i in range(8):
            pltpu.make_async_copy(b_hbm.at[pl.ds(base, _CH)], bidx_v.at[i], sem).wait()
            pltpu.make_async_copy(ev_hbm.at[pl.ds(base, _CH)], bval_v.at[i], sem).wait()
        for i in range(8):
            for g in range(_CH // 16):
                e = bval_v[i, pl.ds(g * 16, 16)]
                bv = bidx_v[i, pl.ds(g * 16, 16)]
                bidx_v[i, pl.ds(g * 16, 16)] = jnp.where(e > 0, bv, -1)
            pltpu.async_copy(
                bval_v.at[i], sh.at[plsc.Indices(bidx_v.at[i], ignored_value=-1)], sem
            )
        for i in range(8):
            pltpu.make_async_copy(
                bval_v.at[i], sh.at[plsc.Indices(bidx_v.at[i], ignored_value=-1)], sem
            ).wait()

    plsc.subcore_barrier()
    pltpu.async_copy(
        sh.at[pl.ds(colbase, B // _NS)],
        part_hbm.at[c].at[pl.ds(colbase, B // _NS)],
        sem,
    ).wait()


def _make_gather(n_rows, n_src):
    nw = n_rows // (_NC * _NS)
    nch = nw // _CH
    step = min(5, nch)
    while nch % step:
        step -= 1

    @functools.partial(
        pl.kernel,
        out_type=jax.ShapeDtypeStruct((n_rows, H), jnp.float32),
        mesh=_SC_MESH,
        scratch_types=[
            pltpu.VMEM((step, _CH, H), jnp.float32),
            pltpu.VMEM((nw,), jnp.int32),
            pltpu.SemaphoreType.DMA,
            pltpu.SemaphoreType.DMA,
        ],
    )
    def _gather_kernel(src_hbm, idx_hbm, out_hbm, rows_v, idx_v, gsem, wsem):
        c = lax.axis_index("c")
        s = lax.axis_index("s")
        wid = s * _NC + c
        base0 = wid * nw
        pltpu.async_copy(idx_hbm.at[pl.ds(base0, nw)], idx_v, gsem).wait()

        @pl.loop(0, nch, step=step)
        def _chunk(k0):
            @pl.when(k0 > 0)
            def _():
                for b in range(step):
                    pltpu.make_async_copy(
                        rows_v.at[b], out_hbm.at[pl.ds(base0, _CH)], wsem
                    ).wait()

            gds = []
            for b in range(step):
                gds.append(pltpu.async_copy(
                    src_hbm.at[plsc.Indices(idx_v.at[pl.ds((k0 + b) * _CH, _CH)])],
                    rows_v.at[b],
                    gsem,
                ))
            for gd in gds:
                gd.wait()
            for b in range(step):
                pltpu.async_copy(
                    rows_v.at[b],
                    out_hbm.at[pl.ds(base0 + (k0 + b) * _CH, _CH)],
                    wsem,
                )

        for b in range(step):
            pltpu.make_async_copy(
                rows_v.at[b], out_hbm.at[pl.ds(base0, _CH)], wsem
            ).wait()

    return _gather_kernel


_gather_n = _make_gather(N, B)
_gather_b = _make_gather(B, N)


def _cummax_body(p_ref, o_ref):
    x = p_ref[0] + p_ref[1]
    li = lax.broadcasted_iota(jnp.int32, (128, 128), 1)
    ri = lax.broadcasted_iota(jnp.int32, (128, 128), 0)
    for k in (1, 2, 4, 8, 16, 32, 64):
        x = jnp.maximum(x, jnp.where(li >= k, pltpu.roll(x, k, 1), 0))
    col = jnp.max(x, axis=1, keepdims=True)
    y = jnp.where(ri >= 1, pltpu.roll(jnp.broadcast_to(col, (128, 128)), 1, 0), 0)
    for k in (1, 2, 4, 8, 16, 32, 64):
        y = jnp.maximum(y, jnp.where(ri >= k, pltpu.roll(y, k, 0), 0))
    o_ref[...] = jnp.maximum(x, y)


def _cummax(part3d):
    return pl.pallas_call(
        _cummax_body,
        out_shape=jax.ShapeDtypeStruct((128, 128), jnp.int32),
    )(part3d)


_SB = 2048


def _t1_body(v_ref, w_ref, b_ref, o_ref):
    o_ref[...] = (
        jnp.dot(
            v_ref[...].astype(jnp.bfloat16),
            w_ref[...].astype(jnp.bfloat16),
            preferred_element_type=jnp.float32,
        )
        + b_ref[...]
    )


def _t1_mm(v_n, W1t, b1row):
    return pl.pallas_call(
        _t1_body,
        grid=(B // _SB,),
        in_specs=[
            pl.BlockSpec((_SB, H), lambda i: (i, 0)),
            pl.BlockSpec((H, H), lambda i: (0, 0)),
            pl.BlockSpec((1, H), lambda i: (0, 0)),
        ],
        out_specs=pl.BlockSpec((_SB, H), lambda i: (i, 0)),
        out_shape=jax.ShapeDtypeStruct((B, H), jnp.float32),
    )(v_n, W1t, b1row)


def _sh_body(v_ref, g_ref, wa_ref, wb_ref, b_ref, o_ref):
    o_ref[...] = (
        jnp.dot(
            v_ref[...].astype(jnp.bfloat16),
            wa_ref[...].astype(jnp.bfloat16),
            preferred_element_type=jnp.float32,
        )
        + jnp.dot(
            g_ref[...].astype(jnp.bfloat16),
            wb_ref[...].astype(jnp.bfloat16),
            preferred_element_type=jnp.float32,
        )
        + b_ref[...]
    )


def _sh_mm(v_n, s_g, W3at, W3bt, b3row):
    return pl.pallas_call(
        _sh_body,
        grid=(B // _SB,),
        in_specs=[
            pl.BlockSpec((_SB, H), lambda i: (i, 0)),
            pl.BlockSpec((_SB, H), lambda i: (i, 0)),
            pl.BlockSpec((H, H), lambda i: (0, 0)),
            pl.BlockSpec((H, H), lambda i: (0, 0)),
            pl.BlockSpec((1, H), lambda i: (0, 0)),
        ],
        out_specs=pl.BlockSpec((_SB, H), lambda i: (i, 0)),
        out_shape=jax.ShapeDtypeStruct((B, H), jnp.float32),
    )(v_n, s_g, W3at, W3bt, b3row)


def kernel(node_embedding, batch, num_count, W1_w, W1_b, W2_w, W2_b, q_w, q_b, W3_w, W3_b):
    batch32 = batch.astype(jnp.int32)
    nxt = jnp.concatenate([batch32[1:], jnp.full((1,), -1, jnp.int32)])
    endv = jnp.where(batch32 != nxt, jnp.arange(N, dtype=jnp.int32) + 1, 0)
    part = _endscan_kernel(batch32, endv)
    cnt = _cummax(part.reshape(_NC, 128, 128)).reshape(B)
    ends = jnp.clip(cnt - 1, 0, N - 1)
    v_n = _gather_b(node_embedding, ends)
    t1 = _t1_mm(v_n, W1_w.T, W1_b.reshape(1, H))
    t1_rep = _gather_n(t1, batch32)
    c = _mid(
        t1_rep,
        node_embedding,
        num_count.reshape(N, 1),
        W2_w.T,
        W2_b.reshape(1, H),
        q_w.reshape(1, H),
        q_b.reshape(1, 1),
    )
    zz = jnp.zeros((_CH, H), jnp.float32)
    cntl = jnp.concatenate(
        [jnp.zeros((1,), jnp.int32), cnt, jnp.full((15,), N, jnp.int32)]
    )
    s_g = jnp.concatenate(
        [_segsum_lo(c, batch32, cntl, zz), _segsum_hi(c, batch32, cntl, zz)]
    )
    s_h = _sh_mm(v_n, s_g, W3_w[:, :H].T, W3_w[:, H:].T, W3_b.reshape(1, H))
    return s_h
